# Initial kernel scaffold; baseline (speedup 1.0000x reference)
import jax
import jax.numpy as jnp
from jax.experimental import pallas as pl


def kernel(items, neighbors, weight_neighbors, seq_hidden_local, mask_item, pos_before_idx, pos_after_idx, embedding_table, pos_before_table, pos_after_table, pos_io_table):
    raise NotImplementedError("write your pallas kernel here")


# speedup vs baseline: 6.3522x; 6.3522x over previous
import functools

import jax
import jax.numpy as jnp
from jax import lax
from jax.experimental import pallas as pl
from jax.experimental.pallas import tpu as pltpu
import jax.experimental.pallas.tpu_sc as plsc

B, L, N, D, V, P = 1024, 50, 12, 64, 100000, 200

NW = 32
NB_ROWS = B * L * N
H_ROWS = B * L
NB_PER_W = NB_ROWS // NW
H_PER_W = H_ROWS // NW
NB_CHUNK = 120
NB_NCHUNK = NB_PER_W // NB_CHUNK
NB_RING = 8
NB_GROUPS = NB_NCHUNK // NB_RING
H_CHUNK = 80
H_NCHUNK = H_PER_W // H_CHUNK
H_RING = 4


def _pair_table_body(pb_ref, pa_ref, pio_ref, out_ref):
    comb = (pb_ref[...][:, None, :] + pa_ref[...][None, :, :]
            + pio_ref[1, :][None, None, :])
    out_ref[...] = comb.reshape(8 * P, D)


def _build_pair_table(pb, pa, pio):
    return pl.pallas_call(
        _pair_table_body,
        grid=(P // 8,),
        in_specs=[
            pl.BlockSpec((8, D), lambda i: (i, 0)),
            pl.BlockSpec((P, D), lambda i: (0, 0)),
            pl.BlockSpec((2, D), lambda i: (0, 0)),
        ],
        out_specs=pl.BlockSpec((8 * P, D), lambda i: (i, 0)),
        out_shape=jax.ShapeDtypeStruct((P * P, D), jnp.float32),
    )(pb, pa, pio)


def _sc_gather_body(emb_hbm, pair_hbm, idx_nb_hbm, idx_pr_hbm, idx_h_hbm,
                    feat_hbm, h_hbm,
                    idx_nb_v, idx_pr_v, idx_h_v, bufs, gsem, asem, wsem):
    wid = lax.axis_index("s") * 2 + lax.axis_index("c")

    pltpu.sync_copy(idx_nb_hbm.at[wid], idx_nb_v)
    pltpu.sync_copy(idx_pr_hbm.at[wid], idx_pr_v)
    pltpu.sync_copy(idx_h_hbm.at[wid], idx_h_v)

    nb_base = wid * NB_PER_W
    h_base = wid * H_PER_W

    def h_gather(c, b):
        return pltpu.async_copy(
            emb_hbm.at[idx_h_v.at[c]], bufs.at[b, pl.ds(0, H_CHUNK)],
            gsem.at[b])

    def h_write(c, b):
        return pltpu.async_copy(
            bufs.at[b, pl.ds(0, H_CHUNK)],
            h_hbm.at[pl.ds(h_base + c * H_CHUNK, H_CHUNK)],
            wsem.at[b])

    for c in range(H_RING):
        h_gather(c, c)
    for c in range(H_NCHUNK):
        b = c % H_RING
        pltpu.make_async_copy(emb_hbm.at[idx_h_v.at[c]],
                              bufs.at[b, pl.ds(0, H_CHUNK)], gsem.at[b]).wait()
        h_write(c, b)
        if c + H_RING < H_NCHUNK:
            pltpu.make_async_copy(bufs.at[b, pl.ds(0, H_CHUNK)],
                                  h_hbm.at[pl.ds(h_base + c * H_CHUNK, H_CHUNK)],
                                  wsem.at[b]).wait()
            h_gather(c + H_RING, b)
    for c in range(H_NCHUNK - H_RING, H_NCHUNK):
        b = c % H_RING
        pltpu.make_async_copy(bufs.at[b, pl.ds(0, H_CHUNK)],
                              h_hbm.at[pl.ds(h_base + c * H_CHUNK, H_CHUNK)],
                              wsem.at[b]).wait()

    def nb_gather(c, j):
        return pltpu.async_copy(emb_hbm.at[idx_nb_v.at[c]], bufs.at[j],
                                gsem.at[j])

    for j in range(NB_RING):
        nb_gather(j, j)

    def group(g, _):
        for j in range(NB_RING):
            c = g * NB_RING + j
            pltpu.make_async_copy(emb_hbm.at[idx_nb_v.at[c]], bufs.at[j],
                                  gsem.at[j]).wait()
            pltpu.async_copy(pair_hbm.at[idx_pr_v.at[c]], bufs.at[j],
                             asem.at[j], add=True)
        for j in range(NB_RING):
            c = g * NB_RING + j
            pltpu.make_async_copy(pair_hbm.at[idx_pr_v.at[c]], bufs.at[j],
                                  asem.at[j]).wait()
            pltpu.async_copy(bufs.at[j],
                             feat_hbm.at[pl.ds(nb_base + c * NB_CHUNK,
                                               NB_CHUNK)],
                             wsem.at[j])
        for j in range(NB_RING):
            c = g * NB_RING + j
            pltpu.make_async_copy(bufs.at[j],
                                  feat_hbm.at[pl.ds(nb_base + c * NB_CHUNK,
                                                    NB_CHUNK)],
                                  wsem.at[j]).wait()

            @pl.when(g < NB_GROUPS - 1)
            def _():
                nb_gather(g * NB_RING + NB_RING + j, j)

        return 0

    lax.fori_loop(0, NB_GROUPS, group, 0)


def _sc_gather(emb, pair_tab, idx_nb, idx_pr, idx_h):
    mesh = plsc.VectorSubcoreMesh(core_axis_name="c", subcore_axis_name="s")
    fn = pl.kernel(
        _sc_gather_body,
        out_type=(
            jax.ShapeDtypeStruct((NB_ROWS, D), jnp.float32),
            jax.ShapeDtypeStruct((H_ROWS, D), jnp.float32),
        ),
        mesh=mesh,
        compiler_params=pltpu.CompilerParams(use_tc_tiling_on_sc=False),
        scratch_types=[
            pltpu.VMEM((NB_NCHUNK, NB_CHUNK), jnp.int32),
            pltpu.VMEM((NB_NCHUNK, NB_CHUNK), jnp.int32),
            pltpu.VMEM((H_NCHUNK, H_CHUNK), jnp.int32),
            pltpu.VMEM((NB_RING, NB_CHUNK, D), jnp.float32),
            pltpu.SemaphoreType.DMA((NB_RING,)),
            pltpu.SemaphoreType.DMA((NB_RING,)),
            pltpu.SemaphoreType.DMA((NB_RING,)),
        ],
    )
    return fn(emb, pair_tab, idx_nb, idx_pr, idx_h)


def _epilogue_body(feat_ref, h_ref, seq_ref, mask_ref, w_ref, out_ref):
    mask = mask_ref[...]
    seq = seq_ref[...]
    denom = jnp.sum(mask, axis=1, keepdims=True) + 1e-8
    s = jnp.sum(seq * mask[..., None], axis=1) / denom
    feat = feat_ref[...]
    e = jnp.sum(feat * s[:, None, None, :], axis=-1)
    e = e * w_ref[...]
    alpha = jax.nn.softmax(e, axis=-1)
    agg = jnp.sum(alpha[..., None] * feat, axis=2)
    out_ref[...] = jax.nn.relu(h_ref[...] + agg)


def _epilogue(feat4, h3, seq, mask, w3, bb=8):
    grid = B // bb
    return pl.pallas_call(
        _epilogue_body,
        grid=(grid,),
        in_specs=[
            pl.BlockSpec((bb, L, N, D), lambda p: (p, 0, 0, 0)),
            pl.BlockSpec((bb, L, D), lambda p: (p, 0, 0)),
            pl.BlockSpec((bb, L, D), lambda p: (p, 0, 0)),
            pl.BlockSpec((bb, L), lambda p: (p, 0)),
            pl.BlockSpec((bb, L, N), lambda p: (p, 0, 0)),
        ],
        out_specs=pl.BlockSpec((bb, L, D), lambda p: (p, 0, 0)),
        out_shape=jax.ShapeDtypeStruct((B, L, D), jnp.float32),
    )(feat4, h3, seq, mask, w3)


def kernel(items, neighbors, weight_neighbors, seq_hidden_local, mask_item,
           pos_before_idx, pos_after_idx, embedding_table,
           pos_before_table, pos_after_table, pos_io_table):
    items = items.astype(jnp.int32)
    neighbors = neighbors.astype(jnp.int32)
    pair_idx = (pos_before_idx.astype(jnp.int32) * P
                + pos_after_idx.astype(jnp.int32))

    idx_nb = neighbors.reshape(NW, NB_NCHUNK, NB_CHUNK)
    idx_pr = pair_idx.reshape(NW, NB_NCHUNK, NB_CHUNK)
    idx_h = items.reshape(NW, H_NCHUNK, H_CHUNK)

    pair_tab = _build_pair_table(pos_before_table, pos_after_table,
                                 pos_io_table)
    feat, h = _sc_gather(embedding_table, pair_tab, idx_nb, idx_pr, idx_h)

    out = _epilogue(feat.reshape(B, L, N, D), h.reshape(B, L, D),
                    seq_hidden_local, mask_item, weight_neighbors)
    return out


# speedup vs baseline: 8.0902x; 1.2736x over previous
import functools

import jax
import jax.numpy as jnp
from jax import lax
from jax.experimental import pallas as pl
from jax.experimental.pallas import tpu as pltpu
import jax.experimental.pallas.tpu_sc as plsc

B, L, N, D, V, P = 1024, 50, 12, 64, 100000, 200

NW = 32
NB_ROWS = B * L * N
H_ROWS = B * L
NB_PER_W = NB_ROWS // NW
H_PER_W = H_ROWS // NW
NB_CHUNK = 120
NB_NCHUNK = NB_PER_W // NB_CHUNK
NB_RING = 8
NB_GROUPS = NB_NCHUNK // NB_RING
H_CHUNK = 80
H_NCHUNK = H_PER_W // H_CHUNK
H_RING = 4


def _pair_table_body(pb_ref, pa_ref, pio_ref, out_ref):
    comb = (pb_ref[...][:, None, :] + pa_ref[...][None, :, :]
            + pio_ref[1, :][None, None, :])
    out_ref[...] = comb.reshape(8 * P, D)


def _build_pair_table(pb, pa, pio):
    return pl.pallas_call(
        _pair_table_body,
        grid=(P // 8,),
        in_specs=[
            pl.BlockSpec((8, D), lambda i: (i, 0)),
            pl.BlockSpec((P, D), lambda i: (0, 0)),
            pl.BlockSpec((2, D), lambda i: (0, 0)),
        ],
        out_specs=pl.BlockSpec((8 * P, D), lambda i: (i, 0)),
        out_shape=jax.ShapeDtypeStruct((P * P, D), jnp.float32),
    )(pb, pa, pio)


def _sc_gather_body(emb_hbm, pair_hbm, idx_nb_hbm, idx_pr_hbm, idx_h_hbm,
                    feat_hbm, h_hbm,
                    idx_nb_v, idx_pr_v, idx_h_v, bufs, gsem, asem, wsem):
    wid = lax.axis_index("s") * 2 + lax.axis_index("c")

    pltpu.sync_copy(idx_nb_hbm.at[wid], idx_nb_v)
    pltpu.sync_copy(idx_pr_hbm.at[wid], idx_pr_v)
    pltpu.sync_copy(idx_h_hbm.at[wid], idx_h_v)

    nb_base = wid * NB_PER_W
    h_base = wid * H_PER_W

    def h_gather(c, b):
        return pltpu.async_copy(
            emb_hbm.at[idx_h_v.at[c]], bufs.at[b, pl.ds(0, H_CHUNK)],
            gsem.at[b])

    def h_write(c, b):
        return pltpu.async_copy(
            bufs.at[b, pl.ds(0, H_CHUNK)],
            h_hbm.at[pl.ds(h_base + c * H_CHUNK, H_CHUNK)],
            wsem.at[b])

    for c in range(H_RING):
        h_gather(c, c)
    for c in range(H_NCHUNK):
        b = c % H_RING
        pltpu.make_async_copy(emb_hbm.at[idx_h_v.at[c]],
                              bufs.at[b, pl.ds(0, H_CHUNK)], gsem.at[b]).wait()
        h_write(c, b)
        if c + H_RING < H_NCHUNK:
            pltpu.make_async_copy(bufs.at[b, pl.ds(0, H_CHUNK)],
                                  h_hbm.at[pl.ds(h_base + c * H_CHUNK, H_CHUNK)],
                                  wsem.at[b]).wait()
            h_gather(c + H_RING, b)
    for c in range(H_NCHUNK - H_RING, H_NCHUNK):
        b = c % H_RING
        pltpu.make_async_copy(bufs.at[b, pl.ds(0, H_CHUNK)],
                              h_hbm.at[pl.ds(h_base + c * H_CHUNK, H_CHUNK)],
                              wsem.at[b]).wait()

    def nb_gather(c, j):
        return pltpu.async_copy(emb_hbm.at[idx_nb_v.at[c]], bufs.at[j],
                                gsem.at[j])

    for j in range(NB_RING):
        nb_gather(j, j)

    def group(g, _):
        for j in range(NB_RING):
            c = g * NB_RING + j
            pltpu.make_async_copy(emb_hbm.at[idx_nb_v.at[c]], bufs.at[j],
                                  gsem.at[j]).wait()
            pltpu.async_copy(pair_hbm.at[idx_pr_v.at[c]], bufs.at[j],
                             asem.at[j], add=True)
        for j in range(NB_RING):
            c = g * NB_RING + j
            pltpu.make_async_copy(pair_hbm.at[idx_pr_v.at[c]], bufs.at[j],
                                  asem.at[j]).wait()
            pltpu.async_copy(bufs.at[j],
                             feat_hbm.at[pl.ds(nb_base + c * NB_CHUNK,
                                               NB_CHUNK)],
                             wsem.at[j])
        for j in range(NB_RING):
            c = g * NB_RING + j
            pltpu.make_async_copy(bufs.at[j],
                                  feat_hbm.at[pl.ds(nb_base + c * NB_CHUNK,
                                                    NB_CHUNK)],
                                  wsem.at[j]).wait()

            @pl.when(g < NB_GROUPS - 1)
            def _():
                nb_gather(g * NB_RING + NB_RING + j, j)

        return 0

    lax.fori_loop(0, NB_GROUPS, group, 0)


def _sc_gather(emb, pair_tab, idx_nb, idx_pr, idx_h):
    mesh = plsc.VectorSubcoreMesh(core_axis_name="c", subcore_axis_name="s")
    fn = pl.kernel(
        _sc_gather_body,
        out_type=(
            jax.ShapeDtypeStruct((NB_ROWS, D), jnp.float32),
            jax.ShapeDtypeStruct((H_ROWS, D), jnp.float32),
        ),
        mesh=mesh,
        compiler_params=pltpu.CompilerParams(use_tc_tiling_on_sc=False),
        scratch_types=[
            pltpu.VMEM((NB_NCHUNK, NB_CHUNK), jnp.int32),
            pltpu.VMEM((NB_NCHUNK, NB_CHUNK), jnp.int32),
            pltpu.VMEM((H_NCHUNK, H_CHUNK), jnp.int32),
            pltpu.VMEM((NB_RING, NB_CHUNK, D), jnp.float32),
            pltpu.SemaphoreType.DMA((NB_RING,)),
            pltpu.SemaphoreType.DMA((NB_RING,)),
            pltpu.SemaphoreType.DMA((NB_RING,)),
        ],
    )
    return fn(emb, pair_tab, idx_nb, idx_pr, idx_h)


def _epilogue_body(feat_ref, h_ref, seq_ref, mask_ref, w_ref, out_ref):
    mask = mask_ref[...]
    seq = seq_ref[...]
    denom = jnp.sum(mask, axis=1, keepdims=True) + 1e-8
    s = jnp.sum(seq * mask[..., None], axis=1) / denom
    sb = s[:, None, :]

    feats = [feat_ref[n] for n in range(N)]
    g = [jnp.sum(feats[n] * sb, axis=-1) * w_ref[n]
         for n in range(N)]
    m = g[0]
    for n in range(1, N):
        m = jnp.maximum(m, g[n])
    u = [jnp.exp(g[n] - m) for n in range(N)]
    z = u[0]
    for n in range(1, N):
        z = z + u[n]
    inv = 1.0 / z
    agg = (u[0] * inv)[:, :, None] * feats[0]
    for n in range(1, N):
        agg = agg + (u[n] * inv)[:, :, None] * feats[n]
    out_ref[...] = jax.nn.relu(h_ref[...] + agg)


def _epilogue(featn, h3, seq, mask, wn, bb=8):
    grid = B // bb
    return pl.pallas_call(
        _epilogue_body,
        grid=(grid,),
        in_specs=[
            pl.BlockSpec((N, bb, L, D), lambda p: (0, p, 0, 0)),
            pl.BlockSpec((bb, L, D), lambda p: (p, 0, 0)),
            pl.BlockSpec((bb, L, D), lambda p: (p, 0, 0)),
            pl.BlockSpec((bb, L), lambda p: (p, 0)),
            pl.BlockSpec((N, bb, L), lambda p: (0, p, 0)),
        ],
        out_specs=pl.BlockSpec((bb, L, D), lambda p: (p, 0, 0)),
        out_shape=jax.ShapeDtypeStruct((B, L, D), jnp.float32),
    )(featn, h3, seq, mask, wn)


def kernel(items, neighbors, weight_neighbors, seq_hidden_local, mask_item,
           pos_before_idx, pos_after_idx, embedding_table,
           pos_before_table, pos_after_table, pos_io_table):
    items = items.astype(jnp.int32)
    nbrs_nm = jnp.transpose(neighbors.astype(jnp.int32), (2, 0, 1))
    pair_nm = jnp.transpose(
        pos_before_idx.astype(jnp.int32) * P + pos_after_idx.astype(jnp.int32),
        (2, 0, 1))
    w_nm = jnp.transpose(weight_neighbors, (2, 0, 1))

    idx_nb = nbrs_nm.reshape(NW, NB_NCHUNK, NB_CHUNK)
    idx_pr = pair_nm.reshape(NW, NB_NCHUNK, NB_CHUNK)
    idx_h = items.reshape(NW, H_NCHUNK, H_CHUNK)

    pair_tab = _build_pair_table(pos_before_table, pos_after_table,
                                 pos_io_table)
    feat, h = _sc_gather(embedding_table, pair_tab, idx_nb, idx_pr, idx_h)

    out = _epilogue(feat.reshape(N, B, L, D), h.reshape(B, L, D),
                    seq_hidden_local, mask_item, w_nm)
    return out


# speedup vs baseline: 8.4224x; 1.0411x over previous
import functools

import jax
import jax.numpy as jnp
from jax import lax
from jax.experimental import pallas as pl
from jax.experimental.pallas import tpu as pltpu
import jax.experimental.pallas.tpu_sc as plsc

B, L, N, D, V, P = 1024, 50, 12, 64, 100000, 200

NW = 32
NB_ROWS = B * L * N
H_ROWS = B * L
NB_PER_W = NB_ROWS // NW
H_PER_W = H_ROWS // NW
NB_CHUNK = 120
NB_NCHUNK = NB_PER_W // NB_CHUNK
NB_RING = 8
NB_GROUPS = NB_NCHUNK // NB_RING
H_CHUNK = 80
H_NCHUNK = H_PER_W // H_CHUNK
H_RING = 4


def _pair_table_body(pb_ref, pa_ref, pio_ref, out_ref):
    comb = (pb_ref[...][:, None, :] + pa_ref[...][None, :, :]
            + pio_ref[1, :][None, None, :])
    out_ref[...] = comb.reshape(8 * P, D)


def _build_pair_table(pb, pa, pio):
    return pl.pallas_call(
        _pair_table_body,
        grid=(P // 8,),
        in_specs=[
            pl.BlockSpec((8, D), lambda i: (i, 0)),
            pl.BlockSpec((P, D), lambda i: (0, 0)),
            pl.BlockSpec((2, D), lambda i: (0, 0)),
        ],
        out_specs=pl.BlockSpec((8 * P, D), lambda i: (i, 0)),
        out_shape=jax.ShapeDtypeStruct((P * P, D), jnp.float32),
    )(pb, pa, pio)


def _sc_gather_body(emb_hbm, pair_hbm, idx_nb_hbm, idx_pr_hbm, idx_h_hbm,
                    feat_hbm, h_hbm,
                    idx_nb_v, idx_pr_v, idx_h_v, bufs, gsem, asem, wsem):
    wid = lax.axis_index("s") * 2 + lax.axis_index("c")

    pltpu.sync_copy(idx_nb_hbm.at[wid], idx_nb_v)
    pltpu.sync_copy(idx_pr_hbm.at[wid], idx_pr_v)
    pltpu.sync_copy(idx_h_hbm.at[wid], idx_h_v)

    nb_base = wid * NB_PER_W
    h_base = wid * H_PER_W

    def h_gather(c, b):
        return pltpu.async_copy(
            emb_hbm.at[idx_h_v.at[c]], bufs.at[b, pl.ds(0, H_CHUNK)],
            gsem.at[b])

    def h_write(c, b):
        return pltpu.async_copy(
            bufs.at[b, pl.ds(0, H_CHUNK)],
            h_hbm.at[pl.ds(h_base + c * H_CHUNK, H_CHUNK)],
            wsem.at[b])

    for c in range(H_RING):
        h_gather(c, c)
    for c in range(H_NCHUNK):
        b = c % H_RING
        pltpu.make_async_copy(emb_hbm.at[idx_h_v.at[c]],
                              bufs.at[b, pl.ds(0, H_CHUNK)], gsem.at[b]).wait()
        h_write(c, b)
        if c + H_RING < H_NCHUNK:
            pltpu.make_async_copy(bufs.at[b, pl.ds(0, H_CHUNK)],
                                  h_hbm.at[pl.ds(h_base + c * H_CHUNK, H_CHUNK)],
                                  wsem.at[b]).wait()
            h_gather(c + H_RING, b)
    for c in range(H_NCHUNK - H_RING, H_NCHUNK):
        b = c % H_RING
        pltpu.make_async_copy(bufs.at[b, pl.ds(0, H_CHUNK)],
                              h_hbm.at[pl.ds(h_base + c * H_CHUNK, H_CHUNK)],
                              wsem.at[b]).wait()

    def nb_gather(c, j):
        return pltpu.async_copy(emb_hbm.at[idx_nb_v.at[c]], bufs.at[j],
                                gsem.at[j])

    for j in range(NB_RING):
        nb_gather(j, j)

    def group(g, _):
        for j in range(NB_RING):
            c = g * NB_RING + j
            pltpu.make_async_copy(emb_hbm.at[idx_nb_v.at[c]], bufs.at[j],
                                  gsem.at[j]).wait()
            pltpu.async_copy(pair_hbm.at[idx_pr_v.at[c]], bufs.at[j],
                             asem.at[j], add=True)
        for j in range(NB_RING):
            c = g * NB_RING + j
            pltpu.make_async_copy(pair_hbm.at[idx_pr_v.at[c]], bufs.at[j],
                                  asem.at[j]).wait()
            pltpu.async_copy(bufs.at[j],
                             feat_hbm.at[pl.ds(nb_base + c * NB_CHUNK,
                                               NB_CHUNK)],
                             wsem.at[j])
        for j in range(NB_RING):
            c = g * NB_RING + j
            pltpu.make_async_copy(bufs.at[j],
                                  feat_hbm.at[pl.ds(nb_base + c * NB_CHUNK,
                                                    NB_CHUNK)],
                                  wsem.at[j]).wait()

            @pl.when(g < NB_GROUPS - 1)
            def _():
                nb_gather(g * NB_RING + NB_RING + j, j)

        return 0

    lax.fori_loop(0, NB_GROUPS, group, 0)


def _sc_gather(emb, pair_tab, idx_nb, idx_pr, idx_h):
    mesh = plsc.VectorSubcoreMesh(core_axis_name="c", subcore_axis_name="s")
    fn = pl.kernel(
        _sc_gather_body,
        out_type=(
            jax.ShapeDtypeStruct((NB_ROWS, D), jnp.float32),
            jax.ShapeDtypeStruct((H_ROWS, D), jnp.float32),
        ),
        mesh=mesh,
        compiler_params=pltpu.CompilerParams(use_tc_tiling_on_sc=False),
        scratch_types=[
            pltpu.VMEM((NB_NCHUNK, NB_CHUNK), jnp.int32),
            pltpu.VMEM((NB_NCHUNK, NB_CHUNK), jnp.int32),
            pltpu.VMEM((H_NCHUNK, H_CHUNK), jnp.int32),
            pltpu.VMEM((NB_RING, NB_CHUNK, D), jnp.float32),
            pltpu.SemaphoreType.DMA((NB_RING,)),
            pltpu.SemaphoreType.DMA((NB_RING,)),
            pltpu.SemaphoreType.DMA((NB_RING,)),
        ],
    )
    return fn(emb, pair_tab, idx_nb, idx_pr, idx_h)


def _epilogue_body(feat_ref, h_ref, seq_ref, mask_ref, w_ref, out_ref):
    bb = seq_ref.shape[0]
    r = bb * L
    mask = mask_ref[...]
    seq = seq_ref[...]
    denom = jnp.sum(mask, axis=1, keepdims=True) + 1e-8
    s = jnp.sum(seq * mask[..., None], axis=1) / denom
    s_rep = jnp.broadcast_to(s[:, None, :], (bb, L, D)).reshape(r, D)

    feats = [feat_ref[n] for n in range(N)]
    w = w_ref[...]
    g = [jnp.sum(feats[n] * s_rep, axis=-1) * w[:, :, n].reshape(r)
         for n in range(N)]
    m = g[0]
    for n in range(1, N):
        m = jnp.maximum(m, g[n])
    u = [jnp.exp(g[n] - m) for n in range(N)]
    z = u[0]
    for n in range(1, N):
        z = z + u[n]
    inv = 1.0 / z
    agg = (u[0] * inv)[:, None] * feats[0]
    for n in range(1, N):
        agg = agg + (u[n] * inv)[:, None] * feats[n]
    out_ref[...] = jax.nn.relu(h_ref[...] + agg)


def _epilogue(featn, h2, seq, mask, wn, bb=8):
    grid = B // bb
    r = bb * L
    return pl.pallas_call(
        _epilogue_body,
        grid=(grid,),
        in_specs=[
            pl.BlockSpec((N, r, D), lambda p: (0, p, 0)),
            pl.BlockSpec((r, D), lambda p: (p, 0)),
            pl.BlockSpec((bb, L, D), lambda p: (p, 0, 0)),
            pl.BlockSpec((bb, L), lambda p: (p, 0)),
            pl.BlockSpec((bb, L, N), lambda p: (p, 0, 0)),
        ],
        out_specs=pl.BlockSpec((r, D), lambda p: (p, 0)),
        out_shape=jax.ShapeDtypeStruct((B * L, D), jnp.float32),
    )(featn, h2, seq, mask, wn)


def kernel(items, neighbors, weight_neighbors, seq_hidden_local, mask_item,
           pos_before_idx, pos_after_idx, embedding_table,
           pos_before_table, pos_after_table, pos_io_table):
    items = items.astype(jnp.int32)
    nbrs_nm = jnp.transpose(neighbors.astype(jnp.int32), (2, 0, 1))
    pair_nm = jnp.transpose(
        pos_before_idx.astype(jnp.int32) * P + pos_after_idx.astype(jnp.int32),
        (2, 0, 1))

    idx_nb = nbrs_nm.reshape(NW, NB_NCHUNK, NB_CHUNK)
    idx_pr = pair_nm.reshape(NW, NB_NCHUNK, NB_CHUNK)
    idx_h = items.reshape(NW, H_NCHUNK, H_CHUNK)

    pair_tab = _build_pair_table(pos_before_table, pos_after_table,
                                 pos_io_table)
    feat, h = _sc_gather(embedding_table, pair_tab, idx_nb, idx_pr, idx_h)

    out2 = _epilogue(feat.reshape(N, B * L, D), h,
                     seq_hidden_local, mask_item, weight_neighbors)
    return out2.reshape(B, L, D)
